# baseline (speedup 1.0000x reference)
import functools

import jax
import jax.numpy as jnp
from jax import lax
from jax.experimental import pallas as pl
from jax.experimental.pallas import tpu as pltpu
from jax.experimental.pallas import tpu_sc as plsc

_CHUNK = 128


@functools.cache
def _build(V, D, B):
    info = plsc.get_sparse_core_info()
    NC, NS = info.num_cores, info.num_subcores
    NW = NC * NS
    b_per_w = B // NW
    n_ch = b_per_w // _CHUNK
    n_grp = b_per_w // 16
    grp_per_ch = _CHUNK // 16
    mesh = plsc.VectorSubcoreMesh(core_axis_name="c", subcore_axis_name="s")

    @functools.partial(
        pl.kernel,
        mesh=mesh,
        out_type=jax.ShapeDtypeStruct((B, D), jnp.float32),
        scratch_types=[
            pltpu.VMEM((n_ch, _CHUNK), jnp.int32),
            pltpu.VMEM((n_ch, _CHUNK), jnp.int32),
            pltpu.VMEM((b_per_w // 2, 2 * D), jnp.float32),
            pltpu.VMEM((b_per_w, D), jnp.float32),
            pltpu.SemaphoreType.DMA,
            pltpu.SemaphoreType.DMA,
        ],
    )
    def k(ids_hbm, table2_hbm, out_hbm,
          idx_v, pairidx_v, pairs_v, rows_v, sem, sem_i):
        wid = lax.axis_index("s") * NC + lax.axis_index("c")
        base = wid * b_per_w

        pltpu.async_copy(ids_hbm.at[wid], idx_v, sem_i).wait()

        def shift_body(t, _):
            v = idx_v[t // grp_per_ch, pl.ds((t % grp_per_ch) * 16, 16)]
            pairidx_v[t // grp_per_ch, pl.ds((t % grp_per_ch) * 16, 16)] = (
                lax.shift_right_logical(v, 1))
            return 0

        lax.fori_loop(0, n_grp, shift_body, 0)

        half_rows = b_per_w // 2
        for h in range(2):
            for j in range(n_ch // 2):
                jj = h * (n_ch // 2) + j
                pltpu.async_copy(
                    table2_hbm.at[pairidx_v.at[jj]],
                    pairs_v.at[pl.ds(j * _CHUNK, _CHUNK)],
                    sem,
                )
            pltpu.make_async_copy(
                table2_hbm.at[pl.ds(0, half_rows)], pairs_v, sem).wait()

            def sel_body(g, _, h=h):
                gg = h * (n_grp // 2) + g
                subv = lax.bitwise_and(
                    idx_v[gg // grp_per_ch,
                          pl.ds((gg % grp_per_ch) * 16, 16)], 1)
                for i in range(16):
                    row = g * 16 + i
                    pbit = subv[i]

                    @pl.when(pbit == 0)
                    def _():
                        for cc in range(D // 16):
                            rows_v[h * half_rows + row,
                                   pl.ds(cc * 16, 16)] = (
                                pairs_v[row, pl.ds(cc * 16, 16)])

                    @pl.when(pbit != 0)
                    def _():
                        for cc in range(D // 16):
                            rows_v[h * half_rows + row,
                                   pl.ds(cc * 16, 16)] = (
                                pairs_v[row, pl.ds(D + cc * 16, 16)])
                return 0

            lax.fori_loop(0, n_grp // 2, sel_body, 0)

        pltpu.sync_copy(rows_v, out_hbm.at[pl.ds(base, b_per_w)])

    return k


def kernel(ids, table):
    B, = ids.shape
    V, D = table.shape
    info = plsc.get_sparse_core_info()
    NW = info.num_cores * info.num_subcores
    b_per_w = B // NW
    ids3 = ids.astype(jnp.int32).reshape(NW, b_per_w // _CHUNK, _CHUNK)
    table2 = table.reshape(V // 2, 2 * D)
    return _build(V, D, B)(ids3, table2)


# speedup vs baseline: 2.5828x; 2.5828x over previous
import functools

import jax
import jax.numpy as jnp
from jax import lax
from jax.experimental import pallas as pl
from jax.experimental.pallas import tpu as pltpu
from jax.experimental.pallas import tpu_sc as plsc


@functools.cache
def _build(V, D, B):
    info = plsc.get_sparse_core_info()
    NC, NS = info.num_cores, info.num_subcores
    NW = NC * NS
    b_per_w = B // NW
    n_grp = b_per_w // 16
    half = V // 2
    mesh = plsc.VectorSubcoreMesh(core_axis_name="c", subcore_axis_name="s")

    @functools.partial(
        pl.kernel,
        mesh=mesh,
        out_type=jax.ShapeDtypeStruct((B, D), jnp.float32),
        scratch_types=[
            pltpu.VMEM((b_per_w,), jnp.int32),
            pltpu.VMEM((b_per_w, D), jnp.float32),
            pltpu.SemaphoreType.DMA,
            pltpu.SemaphoreType.DMA,
        ],
    )
    def k(ids_hbm, table3_hbm, out_hbm, idx_v, rows_v, sem, sem_i):
        wid = lax.axis_index("s") * NC + lax.axis_index("c")
        base = wid * b_per_w
        table_hbm = table3_hbm.reshape(2 * half, D)

        pltpu.async_copy(ids_hbm.at[wid], idx_v, sem_i).wait()

        def fire_body(g, _):
            idvec = idx_v[pl.ds(g * 16, 16)]
            for i in range(16):
                r = idvec[i]
                pltpu.async_copy(
                    table_hbm.at[pl.ds(r, 1)],
                    rows_v.at[pl.ds(g * 16 + i, 1)],
                    sem,
                )
            return 0

        lax.fori_loop(0, n_grp, fire_body, 0)

        pltpu.make_async_copy(
            out_hbm.at[pl.ds(base, b_per_w)], rows_v, sem).wait()

        pltpu.sync_copy(rows_v, out_hbm.at[pl.ds(base, b_per_w)])

    return k


def kernel(ids, table):
    B, = ids.shape
    V, D = table.shape
    info = plsc.get_sparse_core_info()
    NW = info.num_cores * info.num_subcores
    ids2 = ids.astype(jnp.int32).reshape(NW, B // NW)
    table3 = table.reshape(2, V // 2, D)
    return _build(V, D, B)(ids2, table3)
